# Initial kernel scaffold; baseline (speedup 1.0000x reference)
import jax
import jax.numpy as jnp
from jax.experimental import pallas as pl


def kernel(source, target, source_feat, target_feat):
    raise NotImplementedError("write your pallas kernel here")


# speedup vs baseline: 15.0969x; 15.0969x over previous
import functools

import jax
import jax.numpy as jnp
from jax.experimental import pallas as pl

_TILE = 256
_K = 10


def _body(q_ref, k_ref, pos_ref, out_ref):
    q = q_ref[0]
    k = k_ref[0, 0]
    pos = pos_ref[0, 0]

    qn = q * jax.lax.rsqrt(jnp.sum(q * q, axis=1, keepdims=True))
    kn = k * jax.lax.rsqrt(jnp.sum(k * k, axis=1, keepdims=True))
    p = jax.lax.dot_general(
        qn, kn, (((1,), (1,)), ((), ())),
        preferred_element_type=jnp.float32,
    )

    n = p.shape[1]
    iota = jax.lax.broadcasted_iota(jnp.int32, p.shape, 1)
    v0 = jnp.max(p, axis=1)
    w_acc = jnp.zeros_like(p)
    for _ in range(_K):
        vmax = jnp.max(p, axis=1)
        w = jnp.exp(vmax - v0)
        mask = p == vmax[:, None]
        idx = jnp.min(jnp.where(mask, iota, n), axis=1)
        sel = iota == idx[:, None]
        w_acc = w_acc + jnp.where(sel, w[:, None], 0.0)
        p = jnp.where(sel, -jnp.inf, p)

    res = jax.lax.dot_general(
        w_acc, pos, (((1,), (1,)), ((), ())),
        preferred_element_type=jnp.float32,
        precision=jax.lax.Precision.HIGHEST,
    )
    out_ref[0] = res[:, :3] / res[:, 3:4]


@jax.jit
def kernel(source, target, source_feat, target_feat):
    b, n, f = source_feat.shape
    q = jnp.concatenate([target_feat, source_feat], axis=1)
    keys = jnp.stack([source_feat, target_feat], axis=1)
    pos = jnp.stack([source, target], axis=1)
    pos_t = jnp.swapaxes(pos, 2, 3)
    pos_aug = jnp.concatenate(
        [pos_t, jnp.ones((b, 2, 1, n), jnp.float32)], axis=2)

    nt = n // _TILE
    return pl.pallas_call(
        _body,
        grid=(b, 2, nt),
        in_specs=[
            pl.BlockSpec((1, _TILE, f), lambda bi, s, i: (bi, s * nt + i, 0)),
            pl.BlockSpec((1, 1, n, f), lambda bi, s, i: (bi, s, 0, 0)),
            pl.BlockSpec((1, 1, 4, n), lambda bi, s, i: (bi, s, 0, 0)),
        ],
        out_specs=pl.BlockSpec((1, _TILE, 3), lambda bi, s, i: (bi, s * nt + i, 0)),
        out_shape=jax.ShapeDtypeStruct((b, 2 * n, 3), jnp.float32),
    )(q, keys, pos_aug)
